# Initial kernel scaffold; baseline (speedup 1.0000x reference)
import jax
import jax.numpy as jnp
from jax.experimental import pallas as pl


def kernel(points):
    raise NotImplementedError("write your pallas kernel here")


# speedup vs baseline: 3.6395x; 3.6395x over previous
import functools

import jax
import jax.numpy as jnp
import numpy as np
from jax import lax
from jax.experimental import pallas as pl
from jax.experimental.pallas import tpu as pltpu
from jax.experimental.pallas import tpu_sc as plsc

GX, GY = 432, 496
SENT = GX * GY
NCELLS = SENT + 1
MAXV, MAXP = 16000, 32
NPTS = 200000
NPAD = 200704
KROWS, KCOLS = 1568, 128

W = 16
PPW = NPAD // W
NVREG = PPW // 16
CPW = 13408
NCV = CPW // 16
DEAD = 214527
BIGD = 1 << 28

VROWS = 512064
DUMPV = 512032
ZCH = 2048
CROWS = 16016
DUMPC = 16008
FLUSH = 112


def _tc_keys_body(xs_ref, ys_ref, zs_ref, keys_ref):
    x = xs_ref[...]
    y = ys_ref[...]
    z = zs_ref[...]
    xi = jnp.floor((x - jnp.float32(0.0)) / jnp.float32(0.16)).astype(jnp.int32)
    yi = jnp.floor((y - jnp.float32(-39.68)) / jnp.float32(0.16)).astype(jnp.int32)
    zi = jnp.floor((z - jnp.float32(-3.0)) / jnp.float32(4.0)).astype(jnp.int32)
    inr = ((xi >= 0) & (xi < GX) & (yi >= 0) & (yi < GY)
           & (zi >= 0) & (zi < 1))
    key = jnp.where(inr, xi * GY + yi, SENT)
    r = lax.broadcasted_iota(jnp.int32, (KROWS, KCOLS), 0)
    c = lax.broadcasted_iota(jnp.int32, (KROWS, KCOLS), 1)
    key = jnp.where(r * KCOLS + c >= NPTS, DEAD, key)
    keys_ref[...] = key


_tc_keys = pl.pallas_call(
    _tc_keys_body,
    out_shape=jax.ShapeDtypeStruct((KROWS, KCOLS), jnp.int32),
)


def _sc_body(pts8, keys, zsrc, cfill, dumpv, dumpc,
             vox8, crow,
             kbuf, cnt, dense, prow8, crowb, vidxb, gidxb, cidxb,
             tvec, tall, totals_sh, sems):
    w = lax.axis_index("s")
    lo_w = w * CPW
    hi_w = lo_w + CPW
    iota = lax.iota(jnp.int32, 16)
    z16 = jnp.zeros((16,), jnp.int32)

    vbase = w * (VROWS // W)
    d0 = [pltpu.async_copy(zsrc, vox8.at[pl.ds(vbase + i * ZCH, ZCH)],
                           sems.at[0]) for i in range(15)]
    d0.append(pltpu.async_copy(zsrc.at[pl.ds(0, 1284)],
                               vox8.at[pl.ds(vbase + 15 * ZCH, 1284)],
                               sems.at[0]))
    cbase = w * (CROWS // W)
    d1 = [pltpu.async_copy(cfill, crow.at[pl.ds(cbase + i * 128, 128)],
                           sems.at[1]) for i in range(7)]
    d1.append(pltpu.async_copy(cfill.at[pl.ds(0, 105)],
                               crow.at[pl.ds(cbase + 7 * 128, 105)],
                               sems.at[1]))
    pltpu.sync_copy(dumpv, vidxb)
    pltpu.sync_copy(dumpc, cidxb)
    pltpu.sync_copy(dumpc, gidxb)
    for d in d0 + d1:
        d.wait()

    def zero_body(i, _):
        cnt[pl.ds(i * 16, 16)] = z16
        return 0
    lax.fori_loop(0, NCV, zero_body, 0)

    for ch in range(W):
        pltpu.sync_copy(keys.at[pl.ds(ch * PPW, PPW)], kbuf)

        def cnt_body(i, _):
            k = kbuf[pl.ds(i * 16, 16)]
            m = (k >= lo_w) & (k < hi_w)
            rel = jnp.where(m, k - lo_w, 0)
            c, lastm = plsc.scan_count(rel, m)
            base = plsc.load_gather(cnt, [rel], mask=m)
            plsc.store_scatter(cnt, [rel], base + c, mask=m & lastm)
            return 0
        lax.fori_loop(0, NVREG, cnt_body, 0)

    my_n = jnp.clip(NCELLS - lo_w, 0, CPW)

    def tot_body(i, acc):
        x = cnt[pl.ds(i * 16, 16)]
        occ = (x > 0) & (i * 16 + iota < my_n)
        return acc + jnp.sum(occ.astype(jnp.int32))
    total = lax.fori_loop(0, NCV, tot_body, jnp.int32(0))
    tvec[...] = jnp.full((16,), total, jnp.int32)
    pltpu.sync_copy(tvec.at[pl.ds(0, 8)], totals_sh.at[pl.ds(w * 8, 8)])
    plsc.subcore_barrier()
    pltpu.sync_copy(totals_sh, tall)
    tot = plsc.load_gather(tall, [iota * 8])
    base_w = jnp.sum(jnp.where(iota < w, tot, 0))

    def dense_body(i, carry):
        run, nacc = carry
        x = cnt[pl.ds(i * 16, 16)]
        cellv = lo_w + i * 16 + iota
        occ = (x > 0) & (cellv < NCELLS)
        oi = occ.astype(jnp.int32)
        cum = plsc.cumsum(oi)
        densev = run + cum - oi
        dense[pl.ds(i * 16, 16)] = jnp.where(occ, densev, BIGD)
        e = occ & (densev < MAXV)
        ei = e.astype(jnp.int32)
        pos = nacc + plsc.cumsum(ei) - 1
        gx = cellv // GY
        gy = cellv - gx * GY
        iss = cellv == SENT
        gyv = jnp.where(iss, GY, gy)
        gzv = iss.astype(jnp.int32)
        plsc.store_scatter(crowb, [pos, z16], gx, mask=e)
        plsc.store_scatter(crowb, [pos, z16 + 1], gyv, mask=e)
        plsc.store_scatter(crowb, [pos, z16 + 2], gzv, mask=e)
        plsc.store_scatter(crowb, [pos, z16 + 3], jnp.minimum(x, MAXP), mask=e)
        plsc.store_scatter(cidxb, [z16, pos], densev, mask=e)
        nacc2 = nacc + jnp.sum(ei)
        do_flush = nacc2 >= FLUSH

        @pl.when(do_flush)
        def _():
            pltpu.sync_copy(crowb, crow.at[cidxb.at[0]])
            pltpu.sync_copy(dumpc, cidxb)
        return run + jnp.sum(oi), jnp.where(do_flush, 0, nacc2)

    _, nacc = lax.fori_loop(0, NCV, dense_body, (base_w, jnp.int32(0)))

    @pl.when(nacc > 0)
    def _():
        pltpu.sync_copy(crowb, crow.at[cidxb.at[0]])
        pltpu.sync_copy(dumpc, cidxb)

    lax.fori_loop(0, NCV, zero_body, 0)

    def flush_pts():
        pltpu.sync_copy(pts8.at[gidxb.at[0]], prow8)
        pltpu.sync_copy(prow8, vox8.at[vidxb.at[0]])
        pltpu.sync_copy(dumpv, vidxb)

    nacc = jnp.int32(0)
    for ch in range(W):
        pltpu.sync_copy(keys.at[pl.ds(ch * PPW, PPW)], kbuf)

        def pts_body(i, nacc, ch=ch):
            k = kbuf[pl.ds(i * 16, 16)]
            m = (k >= lo_w) & (k < hi_w)
            rel = jnp.where(m, k - lo_w, 0)
            c, lastm = plsc.scan_count(rel, m)
            base = plsc.load_gather(cnt, [rel], mask=m)
            plsc.store_scatter(cnt, [rel], base + c, mask=m & lastm)
            rank = base + c - 1
            dv = plsc.load_gather(dense, [rel], mask=m)
            valid = m & (rank < MAXP) & (dv < MAXV)
            vi = valid.astype(jnp.int32)
            pos = nacc + plsc.cumsum(vi) - 1
            plsc.store_scatter(vidxb, [z16, pos], dv * MAXP + rank, mask=valid)
            plsc.store_scatter(gidxb, [z16, pos], ch * PPW + i * 16 + iota,
                               mask=valid)
            nacc2 = nacc + jnp.sum(vi)
            do_flush = nacc2 >= FLUSH
            pl.when(do_flush)(flush_pts)
            return jnp.where(do_flush, 0, nacc2)
        nacc = lax.fori_loop(0, NVREG, pts_body, nacc)

    pl.when(nacc > 0)(flush_pts)


_sc_mesh = plsc.VectorSubcoreMesh(
    core_axis_name="c", subcore_axis_name="s", num_cores=1)

_sc_vox = pl.kernel(
    _sc_body,
    out_type=[jax.ShapeDtypeStruct((VROWS, 8), jnp.float32),
              jax.ShapeDtypeStruct((CROWS, 8), jnp.int32)],
    mesh=_sc_mesh,
    compiler_params=pltpu.CompilerParams(
        needs_layout_passes=False, use_tc_tiling_on_sc=False),
    scratch_types=[pltpu.VMEM((PPW,), jnp.int32),
                   pltpu.VMEM((CPW,), jnp.int32),
                   pltpu.VMEM((CPW,), jnp.int32),
                   pltpu.VMEM((128, 8), jnp.float32),
                   pltpu.VMEM((128, 8), jnp.int32),
                   pltpu.VMEM((1, 128), jnp.int32),
                   pltpu.VMEM((1, 128), jnp.int32),
                   pltpu.VMEM((1, 128), jnp.int32),
                   pltpu.VMEM((16,), jnp.int32),
                   pltpu.VMEM((128,), jnp.int32),
                   pltpu.VMEM_SHARED((128,), jnp.int32),
                   pltpu.SemaphoreType.DMA((2,))],
)


def kernel(points):
    pts8 = jnp.pad(points, ((0, NPAD - NPTS), (0, 4)))
    soa = jnp.transpose(jnp.pad(points, ((0, NPAD - NPTS), (0, 0))))
    xs = soa[0].reshape(KROWS, KCOLS)
    ys = soa[1].reshape(KROWS, KCOLS)
    zs = soa[2].reshape(KROWS, KCOLS)
    keys = _tc_keys(xs, ys, zs).reshape(NPAD)

    zsrc = np.zeros((ZCH, 8), np.float32)
    cfill = np.broadcast_to(
        np.array([GX, GY, 1, 0, 0, 0, 0, 0], np.int32), (128, 8)).copy()
    dumpv = np.full((1, 128), DUMPV, np.int32)
    dumpc = np.full((1, 128), DUMPC, np.int32)

    vox8, crow = _sc_vox(pts8, keys, zsrc, cfill, dumpv, dumpc)
    voxels = vox8[:MAXV * MAXP, :4].reshape(MAXV, MAXP, 4)
    coordinates = crow[:MAXV, :3]
    num_points_per_voxel = crow[:MAXV, 3]
    return voxels, coordinates, num_points_per_voxel
